# baseline (speedup 1.0000x reference)
import functools

import jax
import jax.numpy as jnp
from jax.experimental import pallas as pl
from jax.experimental.pallas import tpu as pltpu

B, N, J, D = 16, 4096, 21, 256
JP = 32
TOPK = 4
BLK = 4096
NB = N // BLK
COUNT = float(B * N)


def _main_kernel(x1t_ref, x2p_ref, pcl_ref, jf_ref, rel_ref, brel_ref,
                 bmlp_ref, w_ref, out_ref, stats_ref):
    first = (pl.program_id(0) == 0) & (pl.program_id(1) == 0)

    x2 = x2p_ref[0]
    x1t = x1t_ref[0]
    w_d = rel_ref[0]
    b_rel = brel_ref[0]
    dot = jnp.dot(x2.astype(jnp.bfloat16), x1t.astype(jnp.bfloat16),
                  preferred_element_type=jnp.float32)
    x1sq = jnp.zeros((1, BLK), dtype=jnp.float32)
    x2sq = jnp.zeros((JP, 1), dtype=jnp.float32)
    x1w = jnp.zeros((1, BLK), dtype=jnp.float32)
    s2 = jnp.zeros((JP, 1), dtype=jnp.float32)
    for c in range(3):
        x1c = x1t[c:c + 1, :]
        x2c = x2[:, c:c + 1]
        x1sq = x1sq + x1c * x1c
        x2sq = x2sq + x2c * x2c
        wc = rel_ref[c + 1]
        x1w = x1w + x1c * wc
        s2 = s2 + x2c * wc

    row = w_d * x1sq + x1w
    col = w_d * x2sq + (b_rel - s2)
    logit = (-2.0 * w_d) * dot + row
    logit = logit + col
    afac = (jax.nn.sigmoid(logit) * 0.25).astype(jnp.bfloat16)

    iota = jax.lax.broadcasted_iota(jnp.int32, (JP, BLK), 0)
    valid = iota < J
    a_mat = jnp.zeros((JP, BLK), dtype=jnp.bfloat16)
    dkey = -2.0 * dot
    dkey = dkey + x2sq
    d_work = jnp.where(valid, dkey, jnp.inf)
    for k in range(TOPK):
        m = jnp.min(d_work, axis=0, keepdims=True)
        is_min = d_work == m
        a_mat = jnp.where(is_min, afac, a_mat)
        if k + 1 < TOPK:
            d_work = jnp.where(is_min, jnp.inf, d_work)

    p2 = jax.lax.dot_general(
        jf_ref[0], w_ref[:, D:],
        (((1,), (1,)), ((), ())),
        preferred_element_type=jnp.float32)
    x = jax.lax.dot_general(
        pcl_ref[0].astype(jnp.bfloat16), w_ref[:, :D],
        (((1,), (1,)), ((), ())),
        preferred_element_type=jnp.float32)
    x = x + jax.lax.dot_general(
        a_mat, p2.astype(jnp.bfloat16),
        (((0,), (0,)), ((), ())),
        preferred_element_type=jnp.float32)
    x = x + bmlp_ref[0:1, :]
    x_b = x.astype(jnp.bfloat16)
    out_ref[0] = x_b

    ones = jnp.ones((1, BLK), dtype=jnp.bfloat16)
    ssum = jnp.dot(ones, x_b, preferred_element_type=jnp.float32)
    ssq = jnp.dot(ones, x_b * x_b, preferred_element_type=jnp.float32)

    @pl.when(first)
    def _():
        stats_ref[...] = jnp.zeros_like(stats_ref)

    stats_ref[0:1, :] += ssum
    stats_ref[1:2, :] += ssq


def _bn_kernel(x_ref, stats_ref, gamma_ref, beta_ref, out_ref):
    mean = stats_ref[0:1, :] * (1.0 / COUNT)
    ex2 = stats_ref[1:2, :] * (1.0 / COUNT)
    var = ex2 - mean * mean
    inv = jax.lax.rsqrt(var + 1e-5)
    scale = gamma_ref[0:1, :] * inv
    shift = beta_ref[0:1, :] - mean * scale
    out_ref[0] = jnp.maximum(x_ref[0].astype(jnp.float32) * scale + shift, 0.0)


@functools.partial(jax.jit)
def kernel(pcl_feat, joint_feat, pcl_xyz, joint_xyz, W_rel, b_rel,
           W_mlp, b_mlp, gamma, beta):
    x1t = pcl_xyz.transpose(0, 2, 1)
    x2p = jnp.pad(joint_xyz, ((0, 0), (0, JP - J), (0, 0)))
    jf_p = jnp.pad(joint_feat, ((0, 0), (0, JP - J), (0, 0))
                   ).astype(jnp.bfloat16)
    w_b = W_mlp.astype(jnp.bfloat16)

    x, stats = pl.pallas_call(
        _main_kernel,
        grid=(B, NB),
        in_specs=[
            pl.BlockSpec((1, 3, BLK), lambda b, n: (b, 0, n)),
            pl.BlockSpec((1, JP, 3), lambda b, n: (b, 0, 0)),
            pl.BlockSpec((1, BLK, D), lambda b, n: (b, n, 0)),
            pl.BlockSpec((1, JP, D), lambda b, n: (b, 0, 0)),
            pl.BlockSpec(memory_space=pltpu.SMEM),
            pl.BlockSpec(memory_space=pltpu.SMEM),
            pl.BlockSpec((1, D), lambda b, n: (0, 0)),
            pl.BlockSpec((D, 2 * D), lambda b, n: (0, 0)),
        ],
        out_specs=[
            pl.BlockSpec((1, BLK, D), lambda b, n: (b, n, 0)),
            pl.BlockSpec((8, D), lambda b, n: (0, 0)),
        ],
        out_shape=[
            jax.ShapeDtypeStruct((B, N, D), jnp.bfloat16),
            jax.ShapeDtypeStruct((8, D), jnp.float32),
        ],
    )(x1t, x2p, pcl_feat, jf_p, W_rel, b_rel.reshape(1),
      b_mlp.reshape(1, D), w_b)

    BLK2 = 1024
    y = pl.pallas_call(
        _bn_kernel,
        grid=(B, N // BLK2),
        in_specs=[
            pl.BlockSpec((1, BLK2, D), lambda b, n: (b, n, 0)),
            pl.BlockSpec((8, D), lambda b, n: (0, 0)),
            pl.BlockSpec((1, D), lambda b, n: (0, 0)),
            pl.BlockSpec((1, D), lambda b, n: (0, 0)),
        ],
        out_specs=pl.BlockSpec((1, BLK2, D), lambda b, n: (b, n, 0)),
        out_shape=jax.ShapeDtypeStruct((B, N, D), jnp.float32),
    )(x, stats, gamma.reshape(1, D), beta.reshape(1, D))
    return y


# speedup vs baseline: 1.2441x; 1.2441x over previous
import functools

import jax
import jax.numpy as jnp
from jax.experimental import pallas as pl
from jax.experimental.pallas import tpu as pltpu

B, N, J, D = 16, 4096, 21, 256
JP = 32
TOPK = 4
BLK = 4096
NB = N // BLK
COUNT = float(B * N)


def _main_kernel(x1t_ref, x2p_ref, pcl_ref, jf_ref, rel_ref, brel_ref,
                 bmlp_ref, w_ref, out_ref, stats_ref):
    first = (pl.program_id(0) == 0) & (pl.program_id(1) == 0)

    x2 = x2p_ref[0]
    x1t = x1t_ref[0]
    w_d = rel_ref[0]
    b_rel = brel_ref[0]
    dot = jnp.dot(x2.astype(jnp.bfloat16), x1t.astype(jnp.bfloat16),
                  preferred_element_type=jnp.float32)
    x1sq = jnp.zeros((1, BLK), dtype=jnp.float32)
    x2sq = jnp.zeros((JP, 1), dtype=jnp.float32)
    x1w = jnp.zeros((1, BLK), dtype=jnp.float32)
    s2 = jnp.zeros((JP, 1), dtype=jnp.float32)
    for c in range(3):
        x1c = x1t[c:c + 1, :]
        x2c = x2[:, c:c + 1]
        x1sq = x1sq + x1c * x1c
        x2sq = x2sq + x2c * x2c
        wc = rel_ref[c + 1]
        x1w = x1w + x1c * wc
        s2 = s2 + x2c * wc

    row = w_d * x1sq + x1w
    col = w_d * x2sq + (b_rel - s2)
    logit = (-2.0 * w_d) * dot + row
    logit = logit + col
    afac = (jax.nn.sigmoid(logit) * 0.25).astype(jnp.bfloat16)

    iota = jax.lax.broadcasted_iota(jnp.int32, (JP, BLK), 0)
    valid = iota < J
    a_mat = jnp.zeros((JP, BLK), dtype=jnp.bfloat16)
    dkey = -2.0 * dot
    dkey = dkey + x2sq
    d_work = jnp.where(valid, dkey, jnp.inf)
    for k in range(TOPK):
        m = jnp.min(d_work, axis=0, keepdims=True)
        is_min = d_work == m
        a_mat = jnp.where(is_min, afac, a_mat)
        if k + 1 < TOPK:
            d_work = jnp.where(is_min, jnp.inf, d_work)

    p2 = jax.lax.dot_general(
        jf_ref[0], w_ref[:, D:],
        (((1,), (1,)), ((), ())),
        preferred_element_type=jnp.float32)
    x = jax.lax.dot_general(
        pcl_ref[0].astype(jnp.bfloat16), w_ref[:, :D],
        (((1,), (1,)), ((), ())),
        preferred_element_type=jnp.float32)
    x = x + jax.lax.dot_general(
        a_mat, p2.astype(jnp.bfloat16),
        (((0,), (0,)), ((), ())),
        preferred_element_type=jnp.float32)
    x = x + bmlp_ref[0:1, :]
    x_b = x.astype(jnp.bfloat16)
    out_ref[0] = x_b

    ones = jnp.ones((1, BLK), dtype=jnp.bfloat16)
    ssum = jnp.dot(ones, x_b, preferred_element_type=jnp.float32)
    ssq = jnp.dot(ones, x_b * x_b, preferred_element_type=jnp.float32)

    @pl.when(first)
    def _():
        stats_ref[...] = jnp.zeros_like(stats_ref)

    stats_ref[0:1, :] += ssum
    stats_ref[1:2, :] += ssq


def _bn_kernel(x_ref, stats_ref, gamma_ref, beta_ref, out_ref):
    mean = stats_ref[0:1, :] * (1.0 / COUNT)
    ex2 = stats_ref[1:2, :] * (1.0 / COUNT)
    var = ex2 - mean * mean
    inv = jax.lax.rsqrt(var + 1e-5)
    scale = gamma_ref[0:1, :] * inv
    shift = beta_ref[0:1, :] - mean * scale
    out_ref[0] = jnp.maximum(x_ref[0].astype(jnp.float32) * scale + shift, 0.0)


@functools.partial(jax.jit)
def kernel(pcl_feat, joint_feat, pcl_xyz, joint_xyz, W_rel, b_rel,
           W_mlp, b_mlp, gamma, beta):
    x1t = pcl_xyz.transpose(0, 2, 1)
    x2p = jnp.pad(joint_xyz, ((0, 0), (0, JP - J), (0, 0)))
    jf_p = jnp.pad(joint_feat, ((0, 0), (0, JP - J), (0, 0))
                   ).astype(jnp.bfloat16)
    w_b = W_mlp.astype(jnp.bfloat16)

    x, stats = pl.pallas_call(
        _main_kernel,
        grid=(B, NB),
        in_specs=[
            pl.BlockSpec((1, 3, BLK), lambda b, n: (b, 0, n)),
            pl.BlockSpec((1, JP, 3), lambda b, n: (b, 0, 0)),
            pl.BlockSpec((1, BLK, D), lambda b, n: (b, n, 0)),
            pl.BlockSpec((1, JP, D), lambda b, n: (b, 0, 0)),
            pl.BlockSpec(memory_space=pltpu.SMEM),
            pl.BlockSpec(memory_space=pltpu.SMEM),
            pl.BlockSpec((1, D), lambda b, n: (0, 0)),
            pl.BlockSpec((D, 2 * D), lambda b, n: (0, 0)),
        ],
        out_specs=[
            pl.BlockSpec((1, BLK, D), lambda b, n: (b, n, 0)),
            pl.BlockSpec((8, D), lambda b, n: (0, 0)),
        ],
        out_shape=[
            jax.ShapeDtypeStruct((B, N, D), jnp.bfloat16),
            jax.ShapeDtypeStruct((8, D), jnp.float32),
        ],
    )(x1t, x2p, pcl_feat, jf_p, W_rel, b_rel.reshape(1),
      b_mlp.reshape(1, D), w_b)

    BLK2 = 4096
    y = pl.pallas_call(
        _bn_kernel,
        grid=(B, N // BLK2),
        in_specs=[
            pl.BlockSpec((1, BLK2, D), lambda b, n: (b, n, 0)),
            pl.BlockSpec((8, D), lambda b, n: (0, 0)),
            pl.BlockSpec((1, D), lambda b, n: (0, 0)),
            pl.BlockSpec((1, D), lambda b, n: (0, 0)),
        ],
        out_specs=pl.BlockSpec((1, BLK2, D), lambda b, n: (b, n, 0)),
        out_shape=jax.ShapeDtypeStruct((B, N, D), jnp.float32),
    )(x, stats, gamma.reshape(1, D), beta.reshape(1, D))
    return y


# speedup vs baseline: 1.2443x; 1.0002x over previous
import functools

import jax
import jax.numpy as jnp
from jax.experimental import pallas as pl
from jax.experimental.pallas import tpu as pltpu

B, N, J, D = 16, 4096, 21, 256
JP = 24
TOPK = 4
BLK = 4096
NB = N // BLK
COUNT = float(B * N)


def _main_kernel(x1t_ref, x2p_ref, pcl_ref, jf_ref, rel_ref, brel_ref,
                 bmlp_ref, w_ref, out_ref, stats_ref):
    first = (pl.program_id(0) == 0) & (pl.program_id(1) == 0)

    x2 = x2p_ref[0]
    x1t = x1t_ref[0]
    w_d = rel_ref[0]
    b_rel = brel_ref[0]
    dot = jnp.dot(x2.astype(jnp.bfloat16), x1t.astype(jnp.bfloat16),
                  preferred_element_type=jnp.float32)
    x1sq = jnp.zeros((1, BLK), dtype=jnp.float32)
    x2sq = jnp.zeros((JP, 1), dtype=jnp.float32)
    x1w = jnp.zeros((1, BLK), dtype=jnp.float32)
    s2 = jnp.zeros((JP, 1), dtype=jnp.float32)
    for c in range(3):
        x1c = x1t[c:c + 1, :]
        x2c = x2[:, c:c + 1]
        x1sq = x1sq + x1c * x1c
        x2sq = x2sq + x2c * x2c
        wc = rel_ref[c + 1]
        x1w = x1w + x1c * wc
        s2 = s2 + x2c * wc

    row = w_d * x1sq + x1w
    col = w_d * x2sq + (b_rel - s2)
    logit = (-2.0 * w_d) * dot + row
    logit = logit + col
    afac = (jax.nn.sigmoid(logit) * 0.25).astype(jnp.bfloat16)

    iota = jax.lax.broadcasted_iota(jnp.int32, (JP, BLK), 0)
    valid = iota < J
    a_mat = jnp.zeros((JP, BLK), dtype=jnp.bfloat16)
    dkey = -2.0 * dot
    dkey = dkey + x2sq
    d_work = jnp.where(valid, dkey, jnp.inf)
    for k in range(TOPK):
        m = jnp.min(d_work, axis=0, keepdims=True)
        is_min = d_work == m
        a_mat = jnp.where(is_min, afac, a_mat)
        if k + 1 < TOPK:
            d_work = jnp.where(is_min, jnp.inf, d_work)

    p2 = jax.lax.dot_general(
        jf_ref[0], w_ref[:, D:],
        (((1,), (1,)), ((), ())),
        preferred_element_type=jnp.float32)
    x = jax.lax.dot_general(
        pcl_ref[0].astype(jnp.bfloat16), w_ref[:, :D],
        (((1,), (1,)), ((), ())),
        preferred_element_type=jnp.float32)
    x = x + jax.lax.dot_general(
        a_mat, p2.astype(jnp.bfloat16),
        (((0,), (0,)), ((), ())),
        preferred_element_type=jnp.float32)
    x = x + bmlp_ref[0:1, :]
    x_b = x.astype(jnp.bfloat16)
    out_ref[0] = x_b

    ones = jnp.ones((1, BLK), dtype=jnp.bfloat16)
    ssum = jnp.dot(ones, x_b, preferred_element_type=jnp.float32)
    ssq = jnp.dot(ones, x_b * x_b, preferred_element_type=jnp.float32)

    @pl.when(first)
    def _():
        stats_ref[...] = jnp.zeros_like(stats_ref)

    stats_ref[0:1, :] += ssum
    stats_ref[1:2, :] += ssq


def _bn_kernel(x_ref, stats_ref, gamma_ref, beta_ref, out_ref):
    mean = stats_ref[0:1, :] * (1.0 / COUNT)
    ex2 = stats_ref[1:2, :] * (1.0 / COUNT)
    var = ex2 - mean * mean
    inv = jax.lax.rsqrt(var + 1e-5)
    scale = gamma_ref[0:1, :] * inv
    shift = beta_ref[0:1, :] - mean * scale
    out_ref[0] = jnp.maximum(x_ref[0].astype(jnp.float32) * scale + shift, 0.0)


@functools.partial(jax.jit)
def kernel(pcl_feat, joint_feat, pcl_xyz, joint_xyz, W_rel, b_rel,
           W_mlp, b_mlp, gamma, beta):
    x1t = pcl_xyz.transpose(0, 2, 1)
    x2p = jnp.pad(joint_xyz, ((0, 0), (0, JP - J), (0, 0)))
    jf_p = jnp.pad(joint_feat, ((0, 0), (0, JP - J), (0, 0))
                   ).astype(jnp.bfloat16)
    w_b = W_mlp.astype(jnp.bfloat16)

    x, stats = pl.pallas_call(
        _main_kernel,
        grid=(B, NB),
        in_specs=[
            pl.BlockSpec((1, 3, BLK), lambda b, n: (b, 0, n)),
            pl.BlockSpec((1, JP, 3), lambda b, n: (b, 0, 0)),
            pl.BlockSpec((1, BLK, D), lambda b, n: (b, n, 0)),
            pl.BlockSpec((1, JP, D), lambda b, n: (b, 0, 0)),
            pl.BlockSpec(memory_space=pltpu.SMEM),
            pl.BlockSpec(memory_space=pltpu.SMEM),
            pl.BlockSpec((1, D), lambda b, n: (0, 0)),
            pl.BlockSpec((D, 2 * D), lambda b, n: (0, 0)),
        ],
        out_specs=[
            pl.BlockSpec((1, BLK, D), lambda b, n: (b, n, 0)),
            pl.BlockSpec((8, D), lambda b, n: (0, 0)),
        ],
        out_shape=[
            jax.ShapeDtypeStruct((B, N, D), jnp.bfloat16),
            jax.ShapeDtypeStruct((8, D), jnp.float32),
        ],
    )(x1t, x2p, pcl_feat, jf_p, W_rel, b_rel.reshape(1),
      b_mlp.reshape(1, D), w_b)

    BLK2 = 4096
    y = pl.pallas_call(
        _bn_kernel,
        grid=(B, N // BLK2),
        in_specs=[
            pl.BlockSpec((1, BLK2, D), lambda b, n: (b, n, 0)),
            pl.BlockSpec((8, D), lambda b, n: (0, 0)),
            pl.BlockSpec((1, D), lambda b, n: (0, 0)),
            pl.BlockSpec((1, D), lambda b, n: (0, 0)),
        ],
        out_specs=pl.BlockSpec((1, BLK2, D), lambda b, n: (b, n, 0)),
        out_shape=jax.ShapeDtypeStruct((B, N, D), jnp.float32),
    )(x, stats, gamma.reshape(1, D), beta.reshape(1, D))
    return y
